# baseline (speedup 1.0000x reference)
import jax
import jax.numpy as jnp
from jax import lax
from jax.experimental import pallas as pl
from jax.experimental.pallas import tpu as pltpu


def _span_kernel(cand_ref, feat_ref, wemb_ref, out_ref):
    N, _ = cand_ref.shape
    S, H = feat_ref.shape
    NW, Wd = wemb_ref.shape

    cand = cand_ref[...]
    starts = cand[:, 0:1]
    ends = cand[:, 1:2]
    valid = ends > 0
    m_i = valid.astype(jnp.int32)

    s_idx = jnp.clip(starts * m_i, 0, S - 1)
    e_idx = jnp.clip(ends * m_i, 0, S - 1)
    w_idx = jnp.clip(e_idx - s_idx, 0, NW - 1)

    se_idx = jnp.concatenate([s_idx, e_idx], axis=0)
    valid_se = jnp.concatenate([valid, valid], axis=0)
    seq_iota = lax.broadcasted_iota(jnp.int32, (2 * N, S), 1)
    oh_se = ((seq_iota == se_idx) & valid_se).astype(jnp.bfloat16)

    feat = feat_ref[...].astype(jnp.bfloat16)
    se_emb = jnp.dot(oh_se, feat,
                     preferred_element_type=jnp.float32)

    w_iota = lax.broadcasted_iota(jnp.int32, (N, NW), 1)
    oh_w = ((w_iota == w_idx) & valid).astype(jnp.bfloat16)
    wemb = wemb_ref[...].astype(jnp.bfloat16)
    width_emb = jnp.dot(oh_w, wemb,
                        preferred_element_type=jnp.float32)

    out_ref[:, :H] = se_emb[:N, :]
    out_ref[:, H:2 * H] = se_emb[N:, :]
    out_ref[:, 2 * H:] = width_emb


def kernel(features, clause_candidates, width_embedding):
    B, S, H = features.shape
    _, N, _ = clause_candidates.shape
    NW, Wd = width_embedding.shape
    D = 2 * H + Wd

    feat_itemsize = jnp.dtype(features.dtype).itemsize

    cost = pl.CostEstimate(
        flops=2 * B * N * (2 * S * H + NW * Wd),
        transcendentals=0,
        bytes_accessed=(B * S * H * feat_itemsize
                        + B * N * 2 * 4
                        + NW * Wd * 4
                        + B * N * D * feat_itemsize),
    )

    return pl.pallas_call(
        _span_kernel,
        out_shape=jax.ShapeDtypeStruct((B, N, D), features.dtype),
        grid=(B,),
        in_specs=[
            pl.BlockSpec((None, N, 2), lambda b: (b, 0, 0)),
            pl.BlockSpec((None, S, H), lambda b: (b, 0, 0)),
            pl.BlockSpec((NW, Wd), lambda b: (0, 0)),
        ],
        out_specs=pl.BlockSpec((None, N, D), lambda b: (b, 0, 0)),
        compiler_params=pltpu.CompilerParams(
            dimension_semantics=("parallel",),
            vmem_limit_bytes=32 << 20),
        cost_estimate=cost,
    )(clause_candidates.astype(jnp.int32), features, width_embedding)


# speedup vs baseline: 1.6380x; 1.6380x over previous
import jax
import jax.numpy as jnp
from jax import lax
from jax.experimental import pallas as pl
from jax.experimental.pallas import tpu as pltpu


def _span_kernel(cand_ref, feat_ref, wemb_ref, out_ref):
    Bb, N, _ = cand_ref.shape
    _, S, H = feat_ref.shape
    NW, Wd = wemb_ref.shape

    wemb = wemb_ref[...].astype(jnp.bfloat16)

    for b in range(Bb):
        cand = cand_ref[b]
        starts = cand[:, 0:1]
        ends = cand[:, 1:2]
        valid = ends > 0
        m_i = valid.astype(jnp.int32)

        s_idx = jnp.clip(starts * m_i, 0, S - 1)
        e_idx = jnp.clip(ends * m_i, 0, S - 1)
        w_idx = jnp.clip(e_idx - s_idx, 0, NW - 1)

        se_idx = jnp.concatenate([s_idx, e_idx], axis=0)
        valid_se = jnp.concatenate([valid, valid], axis=0)
        seq_iota = lax.broadcasted_iota(jnp.int32, (2 * N, S), 1)
        oh_se = ((seq_iota == se_idx) & valid_se).astype(jnp.bfloat16)

        feat = feat_ref[b].astype(jnp.bfloat16)
        se_emb = jnp.dot(oh_se, feat,
                         preferred_element_type=jnp.float32)

        w_iota = lax.broadcasted_iota(jnp.int32, (N, NW), 1)
        oh_w = ((w_iota == w_idx) & valid).astype(jnp.bfloat16)
        width_emb = jnp.dot(oh_w, wemb,
                            preferred_element_type=jnp.float32)

        out_ref[b, :, :H] = se_emb[:N, :]
        out_ref[b, :, H:2 * H] = se_emb[N:, :]
        out_ref[b, :, 2 * H:] = width_emb


def kernel(features, clause_candidates, width_embedding):
    B, S, H = features.shape
    _, N, _ = clause_candidates.shape
    NW, Wd = width_embedding.shape
    D = 2 * H + Wd

    feat_itemsize = jnp.dtype(features.dtype).itemsize
    block_b = 4

    cost = pl.CostEstimate(
        flops=2 * B * N * (2 * S * H + NW * Wd),
        transcendentals=0,
        bytes_accessed=(B * S * H * feat_itemsize
                        + B * N * 2 * 4
                        + NW * Wd * 4
                        + B * N * D * feat_itemsize),
    )

    return pl.pallas_call(
        _span_kernel,
        out_shape=jax.ShapeDtypeStruct((B, N, D), features.dtype),
        grid=(B // block_b,),
        in_specs=[
            pl.BlockSpec((block_b, N, 2), lambda b: (b, 0, 0)),
            pl.BlockSpec((block_b, S, H), lambda b: (b, 0, 0)),
            pl.BlockSpec((NW, Wd), lambda b: (0, 0)),
        ],
        out_specs=pl.BlockSpec((block_b, N, D), lambda b: (b, 0, 0)),
        compiler_params=pltpu.CompilerParams(
            dimension_semantics=("parallel",),
            vmem_limit_bytes=48 << 20),
        cost_estimate=cost,
    )(clause_candidates.astype(jnp.int32), features, width_embedding)


# speedup vs baseline: 1.6596x; 1.0132x over previous
import jax
import jax.numpy as jnp
from jax import lax
from jax.experimental import pallas as pl
from jax.experimental.pallas import tpu as pltpu


def _span_kernel(cand_ref, feat_ref, wemb_ref, out_ref):
    Bb, N, _ = cand_ref.shape
    _, S, H = feat_ref.shape
    NW, Wd = wemb_ref.shape

    wemb = wemb_ref[...].astype(jnp.bfloat16)

    for b in range(Bb):
        cand = cand_ref[b]
        starts = cand[:, 0:1]
        ends = cand[:, 1:2]
        valid = ends > 0
        m_i = valid.astype(jnp.int32)

        s_idx = jnp.clip(starts * m_i, 0, S - 1)
        e_idx = jnp.clip(ends * m_i, 0, S - 1)
        w_idx = jnp.clip(e_idx - s_idx, 0, NW - 1)

        se_idx = jnp.concatenate([s_idx, e_idx], axis=0)
        valid_se = jnp.concatenate([valid, valid], axis=0)
        seq_iota = lax.broadcasted_iota(jnp.int32, (2 * N, S), 1)
        oh_se = ((seq_iota == se_idx) & valid_se).astype(jnp.bfloat16)

        feat = feat_ref[b].astype(jnp.bfloat16)
        se_emb = jnp.dot(oh_se, feat,
                         preferred_element_type=jnp.float32)

        w_iota = lax.broadcasted_iota(jnp.int32, (N, NW), 1)
        oh_w = ((w_iota == w_idx) & valid).astype(jnp.bfloat16)
        width_emb = jnp.dot(oh_w, wemb,
                            preferred_element_type=jnp.float32)

        out_ref[b, :, :H] = se_emb[:N, :]
        out_ref[b, :, H:2 * H] = se_emb[N:, :]
        out_ref[b, :, 2 * H:] = width_emb


def kernel(features, clause_candidates, width_embedding):
    B, S, H = features.shape
    _, N, _ = clause_candidates.shape
    NW, Wd = width_embedding.shape
    D = 2 * H + Wd

    feat_itemsize = jnp.dtype(features.dtype).itemsize
    block_b = 8

    cost = pl.CostEstimate(
        flops=2 * B * N * (2 * S * H + NW * Wd),
        transcendentals=0,
        bytes_accessed=(B * S * H * feat_itemsize
                        + B * N * 2 * 4
                        + NW * Wd * 4
                        + B * N * D * feat_itemsize),
    )

    return pl.pallas_call(
        _span_kernel,
        out_shape=jax.ShapeDtypeStruct((B, N, D), features.dtype),
        grid=(B // block_b,),
        in_specs=[
            pl.BlockSpec((block_b, N, 2), lambda b: (b, 0, 0)),
            pl.BlockSpec((block_b, S, H), lambda b: (b, 0, 0)),
            pl.BlockSpec((NW, Wd), lambda b: (0, 0)),
        ],
        out_specs=pl.BlockSpec((block_b, N, D), lambda b: (b, 0, 0)),
        compiler_params=pltpu.CompilerParams(
            dimension_semantics=("parallel",),
            vmem_limit_bytes=56 << 20),
        cost_estimate=cost,
    )(clause_candidates.astype(jnp.int32), features, width_embedding)


# speedup vs baseline: 1.6873x; 1.0167x over previous
import jax
import jax.numpy as jnp
from jax import lax
from jax.experimental import pallas as pl
from jax.experimental.pallas import tpu as pltpu


def _span_kernel(cand_ref, feat_ref, wemb_ref, out_ref):
    Bb, N, _ = cand_ref.shape
    _, S, H = feat_ref.shape
    NW, Wd = wemb_ref.shape

    wemb = wemb_ref[...].astype(jnp.bfloat16)

    for b in range(Bb):
        cand = cand_ref[b]
        starts = cand[:, 0:1]
        ends = cand[:, 1:2]
        valid = ends > 0

        s_c = jnp.clip(starts, 0, S - 1)
        e_c = jnp.clip(ends, 0, S - 1)
        s_idx = jnp.where(valid, s_c, -1)
        e_idx = jnp.where(valid, e_c, -1)
        w_idx = jnp.where(valid, jnp.clip(e_c - s_c, 0, NW - 1), -1)

        se_idx = jnp.concatenate([s_idx, e_idx], axis=0)
        seq_iota = lax.broadcasted_iota(jnp.int32, (2 * N, S), 1)
        oh_se = (seq_iota == se_idx).astype(jnp.bfloat16)

        feat = feat_ref[b].astype(jnp.bfloat16)
        se_emb = jnp.dot(oh_se, feat,
                         preferred_element_type=jnp.float32)

        w_iota = lax.broadcasted_iota(jnp.int32, (N, NW), 1)
        oh_w = (w_iota == w_idx).astype(jnp.bfloat16)
        width_emb = jnp.dot(oh_w, wemb,
                            preferred_element_type=jnp.float32)

        out_ref[b, :, :H] = se_emb[:N, :]
        out_ref[b, :, H:2 * H] = se_emb[N:, :]
        out_ref[b, :, 2 * H:] = width_emb


def kernel(features, clause_candidates, width_embedding):
    B, S, H = features.shape
    _, N, _ = clause_candidates.shape
    NW, Wd = width_embedding.shape
    D = 2 * H + Wd

    feat_itemsize = jnp.dtype(features.dtype).itemsize
    block_b = 8

    cost = pl.CostEstimate(
        flops=2 * B * N * (2 * S * H + NW * Wd),
        transcendentals=0,
        bytes_accessed=(B * S * H * feat_itemsize
                        + B * N * 2 * 4
                        + NW * Wd * 4
                        + B * N * D * feat_itemsize),
    )

    return pl.pallas_call(
        _span_kernel,
        out_shape=jax.ShapeDtypeStruct((B, N, D), features.dtype),
        grid=(B // block_b,),
        in_specs=[
            pl.BlockSpec((block_b, N, 2), lambda b: (b, 0, 0)),
            pl.BlockSpec((block_b, S, H), lambda b: (b, 0, 0)),
            pl.BlockSpec((NW, Wd), lambda b: (0, 0)),
        ],
        out_specs=pl.BlockSpec((block_b, N, D), lambda b: (b, 0, 0)),
        compiler_params=pltpu.CompilerParams(
            dimension_semantics=("parallel",),
            vmem_limit_bytes=56 << 20),
        cost_estimate=cost,
    )(clause_candidates.astype(jnp.int32), features, width_embedding)
